# baseline (speedup 1.0000x reference)
import functools
import math

import jax
import jax.numpy as jnp
from jax import lax
from jax.experimental import pallas as pl
from jax.experimental.pallas import tpu as pltpu
from jax.experimental.pallas import tpu_sc as plsc

_VOCAB = 100000
_D = 512
_SCALE = math.sqrt(_D)
_LANES = 16

_NC = 2
_NS = 16
_NW = _NC * _NS

_BATCH = 4096
_SEQ = 50
_B = _BATCH * _SEQ
_B_PER_W = _B // _NW
_CHUNK = 64
_N_CHUNKS = _B_PER_W // _CHUNK
_NBUF = 3
_N_MAIN = (_N_CHUNKS // _NBUF) * _NBUF


def _emb_body(idx_hbm, lut_hbm, out_hbm, idx_v, *scratch):
    bufs = scratch[:_NBUF]
    gsem = scratch[_NBUF:2 * _NBUF]
    ssem = scratch[2 * _NBUF:]
    wid = lax.axis_index("s") * _NC + lax.axis_index("c")
    base = pl.multiple_of(wid * _B_PER_W, _B_PER_W)
    pltpu.sync_copy(idx_hbm.at[pl.ds(base, _B_PER_W)], idx_v)

    def gather(g, k):
        off = pl.multiple_of(g * _CHUNK, _CHUNK)
        pltpu.async_copy(lut_hbm.at[idx_v.at[pl.ds(off, _CHUNK)]], bufs[k],
                         gsem[k])

    for k in range(_NBUF - 1):
        gather(k, k)

    def outer(go, carry):
        for k in range(_NBUF):
            g = go * _NBUF + k
            kn = (k + _NBUF - 1) % _NBUF
            pltpu.make_async_copy(out_hbm.at[pl.ds(0, _CHUNK)], bufs[k],
                                  gsem[k]).wait()

            def row_body(i, c2, _buf=bufs[k]):
                for j in range(_D // _LANES):
                    sl = _buf[i, pl.ds(j * _LANES, _LANES)]
                    _buf[i, pl.ds(j * _LANES, _LANES)] = sl * _SCALE
                return c2

            lax.fori_loop(0, _CHUNK, row_body, 0)

            off = pl.multiple_of(g * _CHUNK, _CHUNK)
            pltpu.async_copy(bufs[k], out_hbm.at[pl.ds(base + off, _CHUNK)],
                             ssem[k])

            if k == 0:
                @pl.when(go == 0)
                def _():
                    gather(_NBUF - 1, kn)

                @pl.when(jnp.logical_and(go >= 1, g + _NBUF - 1 < _N_CHUNKS))
                def _():
                    pltpu.make_async_copy(bufs[kn],
                                          out_hbm.at[pl.ds(0, _CHUNK)],
                                          ssem[kn]).wait()
                    gather(g + _NBUF - 1, kn)
            else:
                @pl.when(g + _NBUF - 1 < _N_CHUNKS)
                def _():
                    pltpu.make_async_copy(bufs[kn],
                                          out_hbm.at[pl.ds(0, _CHUNK)],
                                          ssem[kn]).wait()
                    gather(g + _NBUF - 1, kn)

        return carry

    lax.fori_loop(0, _N_MAIN // _NBUF, outer, 0)

    for g in range(_N_MAIN, _N_CHUNKS):
        k = g % _NBUF
        pltpu.make_async_copy(out_hbm.at[pl.ds(0, _CHUNK)], bufs[k],
                              gsem[k]).wait()

        def tail_row(i, c2, _buf=bufs[k]):
            for j in range(_D // _LANES):
                sl = _buf[i, pl.ds(j * _LANES, _LANES)]
                _buf[i, pl.ds(j * _LANES, _LANES)] = sl * _SCALE
            return c2

        lax.fori_loop(0, _CHUNK, tail_row, 0)
        off = g * _CHUNK
        pltpu.async_copy(bufs[k], out_hbm.at[pl.ds(base + off, _CHUNK)],
                         ssem[k])

    for k in range(_NBUF):
        pltpu.make_async_copy(bufs[k], out_hbm.at[pl.ds(0, _CHUNK)],
                              ssem[k]).wait()


@jax.jit
def _emb(x_flat_t, lut):
    mesh = plsc.VectorSubcoreMesh(core_axis_name="c", subcore_axis_name="s")
    rows = functools.partial(
        pl.kernel,
        mesh=mesh,
        out_type=jax.ShapeDtypeStruct((_B, _D), jnp.float32),
        scratch_types=(
            [pltpu.VMEM((_B_PER_W,), jnp.int32)]
            + [pltpu.VMEM((_CHUNK, _D), jnp.float32) for _ in range(_NBUF)]
            + [pltpu.SemaphoreType.DMA for _ in range(2 * _NBUF)]
        ),
    )(_emb_body)(x_flat_t, lut)
    return rows.reshape(_SEQ, _BATCH, _D).transpose(1, 0, 2)


def kernel(x, lut):
    return _emb(x.astype(jnp.int32).T.reshape(-1), lut)


# speedup vs baseline: 1.0062x; 1.0062x over previous
import functools
import math

import jax
import jax.numpy as jnp
from jax import lax
from jax.experimental import pallas as pl
from jax.experimental.pallas import tpu as pltpu
from jax.experimental.pallas import tpu_sc as plsc

_VOCAB = 100000
_D = 512
_SCALE = math.sqrt(_D)
_LANES = 16

_NC = 2
_NS = 16
_NW = _NC * _NS

_BATCH = 4096
_SEQ = 50
_B = _BATCH * _SEQ
_B_PER_W = _B // _NW
_CHUNK = 40
_N_CHUNKS = _B_PER_W // _CHUNK
_NBUF = 4
_N_MAIN = (_N_CHUNKS // _NBUF) * _NBUF


def _emb_body(idx_hbm, lut_hbm, out_hbm, idx_v, *scratch):
    bufs = scratch[:_NBUF]
    gsem = scratch[_NBUF:2 * _NBUF]
    ssem = scratch[2 * _NBUF:]
    wid = lax.axis_index("s") * _NC + lax.axis_index("c")
    base = pl.multiple_of(wid * _B_PER_W, _B_PER_W)
    pltpu.sync_copy(idx_hbm.at[pl.ds(base, _B_PER_W)], idx_v)

    def gather(g, k):
        off = pl.multiple_of(g * _CHUNK, _CHUNK)
        pltpu.async_copy(lut_hbm.at[idx_v.at[pl.ds(off, _CHUNK)]], bufs[k],
                         gsem[k])

    for k in range(_NBUF - 1):
        gather(k, k)

    def outer(go, carry):
        for k in range(_NBUF):
            g = go * _NBUF + k
            kn = (k + _NBUF - 1) % _NBUF
            pltpu.make_async_copy(out_hbm.at[pl.ds(0, _CHUNK)], bufs[k],
                                  gsem[k]).wait()

            def row_body(i, c2, _buf=bufs[k]):
                for j in range(_D // _LANES):
                    sl = _buf[i, pl.ds(j * _LANES, _LANES)]
                    _buf[i, pl.ds(j * _LANES, _LANES)] = sl * _SCALE
                return c2

            lax.fori_loop(0, _CHUNK, row_body, 0)

            off = pl.multiple_of(g * _CHUNK, _CHUNK)
            pltpu.async_copy(bufs[k], out_hbm.at[pl.ds(base + off, _CHUNK)],
                             ssem[k])

            if k == 0:
                @pl.when(go == 0)
                def _():
                    gather(_NBUF - 1, kn)

                @pl.when(jnp.logical_and(go >= 1, g + _NBUF - 1 < _N_CHUNKS))
                def _():
                    pltpu.make_async_copy(bufs[kn],
                                          out_hbm.at[pl.ds(0, _CHUNK)],
                                          ssem[kn]).wait()
                    gather(g + _NBUF - 1, kn)
            else:
                @pl.when(g + _NBUF - 1 < _N_CHUNKS)
                def _():
                    pltpu.make_async_copy(bufs[kn],
                                          out_hbm.at[pl.ds(0, _CHUNK)],
                                          ssem[kn]).wait()
                    gather(g + _NBUF - 1, kn)

        return carry

    lax.fori_loop(0, _N_MAIN // _NBUF, outer, 0)

    for g in range(_N_MAIN, _N_CHUNKS):
        k = g % _NBUF
        pltpu.make_async_copy(out_hbm.at[pl.ds(0, _CHUNK)], bufs[k],
                              gsem[k]).wait()

        def tail_row(i, c2, _buf=bufs[k]):
            for j in range(_D // _LANES):
                sl = _buf[i, pl.ds(j * _LANES, _LANES)]
                _buf[i, pl.ds(j * _LANES, _LANES)] = sl * _SCALE
            return c2

        lax.fori_loop(0, _CHUNK, tail_row, 0)
        off = g * _CHUNK
        pltpu.async_copy(bufs[k], out_hbm.at[pl.ds(base + off, _CHUNK)],
                         ssem[k])

    for k in range(_NBUF):
        pltpu.make_async_copy(bufs[k], out_hbm.at[pl.ds(0, _CHUNK)],
                              ssem[k]).wait()


@jax.jit
def _emb(x_flat_t, lut):
    mesh = plsc.VectorSubcoreMesh(core_axis_name="c", subcore_axis_name="s")
    rows = functools.partial(
        pl.kernel,
        mesh=mesh,
        out_type=jax.ShapeDtypeStruct((_B, _D), jnp.float32),
        scratch_types=(
            [pltpu.VMEM((_B_PER_W,), jnp.int32)]
            + [pltpu.VMEM((_CHUNK, _D), jnp.float32) for _ in range(_NBUF)]
            + [pltpu.SemaphoreType.DMA for _ in range(2 * _NBUF)]
        ),
    )(_emb_body)(x_flat_t, lut)
    return rows.reshape(_SEQ, _BATCH, _D).transpose(1, 0, 2)


def kernel(x, lut):
    return _emb(x.astype(jnp.int32).T.reshape(-1), lut)
